# baseline (speedup 1.0000x reference)
import functools

import jax
import jax.numpy as jnp
from jax.experimental import pallas as pl
from jax.experimental.pallas import tpu as pltpu

_NUM_GRAPHS = 64
_ROW_BLK = 2000


def _layer_body(agg_ref, rdeg_ref, w_ref, b_ref, o_ref):
    a = agg_ref[...] * rdeg_ref[...]
    acc = jax.lax.dot_general(
        a, w_ref[...], (((1,), (1,)), ((), ())),
        preferred_element_type=jnp.float32,
        precision=jax.lax.Precision.HIGHEST)
    o_ref[...] = jnp.maximum(acc + b_ref[...], 0.0)


def _fused_layer(agg, rdeg2d, w, b2d):
    n, h = agg.shape
    grid = (n // _ROW_BLK,)
    return pl.pallas_call(
        _layer_body,
        grid=grid,
        in_specs=[
            pl.BlockSpec((_ROW_BLK, h), lambda i: (i, 0)),
            pl.BlockSpec((_ROW_BLK, 1), lambda i: (i, 0)),
            pl.BlockSpec((h, h), lambda i: (0, 0)),
            pl.BlockSpec((1, h), lambda i: (0, 0)),
        ],
        out_specs=pl.BlockSpec((_ROW_BLK, h), lambda i: (i, 0)),
        out_shape=jax.ShapeDtypeStruct((n, h), jnp.float32),
    )(agg, rdeg2d, w, b2d)


def _mlp_body(x_ref, w1_ref, b1_ref, w2_ref, o_ref):
    x = x_ref[...]
    h2 = jax.lax.dot_general(
        x, w1_ref[...], (((1,), (1,)), ((), ())),
        preferred_element_type=jnp.float32,
        precision=jax.lax.Precision.HIGHEST)
    h2 = jnp.maximum(h2 + b1_ref[...], 0.0)
    o_ref[...] = jax.lax.dot_general(
        h2, w2_ref[...], (((1,), (1,)), ((), ())),
        preferred_element_type=jnp.float32,
        precision=jax.lax.Precision.HIGHEST)


def _mlp(xcat, fc1W, fc1b2d, fc2W):
    g = xcat.shape[0]
    return pl.pallas_call(
        _mlp_body,
        out_shape=jax.ShapeDtypeStruct((g, 1), jnp.float32),
    )(xcat, fc1W, fc1b2d, fc2W)


def kernel(x, edge_index, edge_weight, batch, emb, convW, convB, fc1W, fc1b, fc2W, fc2b):
    n, h = emb.shape
    src = edge_index[1]
    dst = edge_index[0]
    ew = jnp.abs(edge_weight)
    deg = jnp.zeros((n,), jnp.float32).at[dst].add(1.0)
    rdeg2d = (1.0 / jnp.maximum(deg, 1.0))[:, None]

    hcur = emb[x]
    for i in range(convW.shape[0]):
        msgs = hcur[src] * ew[:, None]
        agg = jnp.zeros((n, h), jnp.float32).at[dst].add(msgs)
        hcur = _fused_layer(agg, rdeg2d, convW[i], convB[i][None, :])

    ones = jnp.ones((n, 1), jnp.float32)
    counts = jax.ops.segment_sum(ones, batch, num_segments=_NUM_GRAPHS)
    sums = jax.ops.segment_sum(hcur, batch, num_segments=_NUM_GRAPHS)
    mean_x = sums / jnp.maximum(counts, 1.0)
    max_x = jax.ops.segment_max(hcur, batch, num_segments=_NUM_GRAPHS)
    max_x = jnp.where(counts > 0, max_x, 0.0)
    xcat = jnp.concatenate([mean_x, max_x], axis=1)

    out = _mlp(xcat, fc1W, fc1b[None, :], fc2W)
    return out[:, 0] + fc2b[0]


# speedup vs baseline: 2.3313x; 2.3313x over previous
import dataclasses
import functools

import jax
import jax.numpy as jnp
from jax import lax
from jax.experimental import pallas as pl
from jax.experimental.pallas import tpu as pltpu
from jax.experimental.pallas import tpu_sc as plsc

_NUM_GRAPHS = 64
_ROW_BLK = 2000
_NPAD = 10240
_EBLK = 512
_GSUB = _EBLK // 128
_SUBCORES = 16


def _sc_mesh():
    return plsc.VectorSubcoreMesh(core_axis_name="c", subcore_axis_name="s")


def _sc_compiler_params():
    cp = pltpu.CompilerParams()
    if "needs_layout_passes" in pltpu.CompilerParams.__dataclass_fields__:
        cp = dataclasses.replace(cp, needs_layout_passes=False)
    return cp


def _emb_gather(embAB, x):
    n = x.shape[0]
    n2 = embAB.shape[0]
    nblk = (n + 127) // 128
    per_tile = (nblk + _SUBCORES - 1) // _SUBCORES

    @functools.partial(
        pl.kernel,
        out_type=jax.ShapeDtypeStruct((n2, 128), jnp.float32),
        mesh=_sc_mesh(),
        scratch_types=[
            pltpu.VMEM((128,), jnp.int32),
            pltpu.VMEM((128, 128), jnp.float32),
            pltpu.SemaphoreType.DMA,
        ],
    )
    def k(emb_hbm, x_hbm, o_hbm, idxv, rows, sem):
        c = lax.axis_index("c")
        s = lax.axis_index("s")
        cn = c * n

        @pl.loop(0, per_tile)
        def _w(w):
            b = s + w * _SUBCORES

            @pl.when(b < nblk)
            def _():
                start = jnp.minimum(b * 128, n - 128)
                pltpu.sync_copy(x_hbm.at[pl.ds(start, 128)], idxv)
                for j in range(8):
                    idxv[pl.ds(j * 16, 16)] = idxv[pl.ds(j * 16, 16)] + cn
                pltpu.async_copy(emb_hbm.at[idxv], rows, sem).wait()
                pltpu.sync_copy(rows, o_hbm.at[pl.ds(cn + start, 128)])

    return k(embAB, x)


def _msg_pass(hAB, src2d, dst2d, ew2d):
    n2 = hAB.shape[0]
    n = n2 // 2
    erows = src2d.shape[0]
    rows_per_tile = erows // _SUBCORES
    nblocks = rows_per_tile // 8

    @functools.partial(
        pl.kernel, out_type=jax.ShapeDtypeStruct((n2, 128), jnp.float32),
        mesh=_sc_mesh(),
        scratch_types=[
            pltpu.VMEM_SHARED((_NPAD, 128), jnp.float32),
            pltpu.VMEM((8, 128), jnp.int32),
            pltpu.VMEM((8, 128), jnp.int32),
            pltpu.VMEM((8, 128), jnp.float32),
            pltpu.VMEM((256, 128), jnp.float32),
            pltpu.SemaphoreType.DMA,
        ],
        compiler_params=_sc_compiler_params(),
    )
    def k(h_hbm, src_hbm, dst_hbm, ew_hbm, agg_out, agg_sh, srcv, dstv, ewv,
          rows, sem):
        c = lax.axis_index("c")
        s = lax.axis_index("s")
        cn = c * n

        @pl.loop(0, 128)
        def _zb(r):
            for j in range(8):
                rows[r, pl.ds(j * 16, 16)] = jnp.zeros((16,), jnp.float32)

        @pl.loop(0, _NPAD // _SUBCORES // 128)
        def _z(i):
            pltpu.sync_copy(
                rows.at[pl.ds(0, 128)],
                agg_sh.at[pl.ds(s * (_NPAD // _SUBCORES) + i * 128, 128)])

        plsc.subcore_barrier()

        @pl.loop(0, nblocks)
        def _blk(b):
            rbase = s * rows_per_tile + b * 8
            pltpu.sync_copy(src_hbm.at[pl.ds(rbase, 8)], srcv)
            pltpu.sync_copy(dst_hbm.at[pl.ds(rbase, 8)], dstv)
            pltpu.sync_copy(ew_hbm.at[pl.ds(rbase, 8)], ewv)
            for g in range(8):
                for j in range(8):
                    srcv[g, pl.ds(j * 16, 16)] = srcv[g, pl.ds(j * 16, 16)] + cn
                    ewv[g, pl.ds(j * 16, 16)] = jnp.abs(ewv[g, pl.ds(j * 16, 16)])
            for half in range(4):
                copies = [
                    pltpu.async_copy(h_hbm.at[srcv.at[half * 2 + g]],
                                     rows.at[pl.ds(g * 128, 128)], sem)
                    for g in range(2)
                ]
                for cp in copies:
                    cp.wait()
                for g in range(2):
                    @pl.loop(0, 128)
                    def _r(r, g=g, half=half):
                        w = plsc.load_gather(
                            ewv, [jnp.full((16,), half * 2 + g, jnp.int32),
                                  jnp.full((16,), r, jnp.int32)])
                        for j in range(8):
                            rows[g * 128 + r, pl.ds(j * 16, 16)] = (
                                rows[g * 128 + r, pl.ds(j * 16, 16)] * w)
                for g in range(2):
                    pltpu.sync_copy(rows.at[pl.ds(g * 128, 128)],
                                    agg_sh.at[dstv.at[half * 2 + g]],
                                    add=True)

        plsc.subcore_barrier()
        rpt = (n // _SUBCORES // 8) * 8

        @pl.when(s < _SUBCORES - 1)
        def _wb_most():
            pltpu.sync_copy(agg_sh.at[pl.ds(s * rpt, rpt)],
                            agg_out.at[pl.ds(cn + s * rpt, rpt)])

        @pl.when(s == _SUBCORES - 1)
        def _wb_last():
            tail = n - (_SUBCORES - 1) * rpt
            base = (_SUBCORES - 1) * rpt
            pltpu.sync_copy(agg_sh.at[pl.ds(base, tail)],
                            agg_out.at[pl.ds(cn + base, tail)])

    return k(hAB, src2d, dst2d, ew2d)


def _deg_hist(dst2d):
    erows = dst2d.shape[0]
    rows_per_core = erows // 2
    rows_per_tile = rows_per_core // _SUBCORES
    nblocks = rows_per_tile // 8

    @functools.partial(
        pl.kernel, out_type=jax.ShapeDtypeStruct((256, 128), jnp.float32),
        mesh=_sc_mesh(),
        scratch_types=[
            pltpu.VMEM_SHARED((128, 128), jnp.float32),
            pltpu.VMEM((8, 128), jnp.int32),
            pltpu.VMEM((128, 128), jnp.float32),
            pltpu.VMEM((1, 128), jnp.int32),
        ],
        compiler_params=_sc_compiler_params(),
    )
    def k(dst_hbm, deg_out, deg_sh, dstv, acc, identv):
        c = lax.axis_index("c")
        s = lax.axis_index("s")

        @pl.loop(0, 128)
        def _z(r):
            for j in range(8):
                acc[r, pl.ds(j * 16, 16)] = jnp.zeros((16,), jnp.float32)
        for j in range(8):
            identv[0, pl.ds(j * 16, 16)] = lax.iota(jnp.int32, 16) + j * 16

        @pl.when(s == 0)
        def _():
            pltpu.sync_copy(acc, deg_sh)

        plsc.subcore_barrier()

        @pl.loop(0, nblocks)
        def _blk(b):
            rbase = c * rows_per_core + s * rows_per_tile + b * 8
            pltpu.sync_copy(dst_hbm.at[pl.ds(rbase, 8)], dstv)

            @pl.loop(0, 8)
            def _row(g):
                for j in range(8):
                    iv = plsc.load_gather(
                        dstv, [jnp.full((16,), g, jnp.int32),
                               lax.iota(jnp.int32, 16) + j * 16])
                    hi = lax.shift_right_logical(iv, 7)
                    lo = jnp.bitwise_and(iv, 127)
                    plsc.addupdate_scatter(
                        acc, [hi, lo], jnp.ones((16,), jnp.float32))

        pltpu.sync_copy(acc, deg_sh.at[identv.at[0]], add=True)
        plsc.subcore_barrier()

        @pl.when(s == 0)
        def _():
            pltpu.sync_copy(deg_sh, deg_out.at[pl.ds(c * 128, 128)])

    return k(dst2d)


def _layer_body(a1_ref, a2_ref, rdeg_ref, w1_ref, w2_ref, b_ref, o_ref):
    rd = rdeg_ref[...]
    acc = jax.lax.dot_general(
        a1_ref[...] * rd, w1_ref[...], (((1,), (1,)), ((), ())),
        preferred_element_type=jnp.float32,
        precision=jax.lax.Precision.HIGHEST)
    acc += jax.lax.dot_general(
        a2_ref[...] * rd, w2_ref[...], (((1,), (1,)), ((), ())),
        preferred_element_type=jnp.float32,
        precision=jax.lax.Precision.HIGHEST)
    o_ref[...] = jnp.maximum(acc + b_ref[...], 0.0)


def _fused_layer(aggAB, rdeg2d, w, b2d, full_out):
    n = rdeg2d.shape[0]
    h = w.shape[0]
    nblk = n // _ROW_BLK
    if full_out:
        out_shape = jax.ShapeDtypeStruct((n, h), jnp.float32)
        out_spec = pl.BlockSpec((_ROW_BLK, 128), lambda i, c: (i, c))
    else:
        out_shape = jax.ShapeDtypeStruct((2 * n, 128), jnp.float32)
        out_spec = pl.BlockSpec((_ROW_BLK, 128), lambda i, c: (c * nblk + i, 0))
    return pl.pallas_call(
        _layer_body,
        grid=(nblk, 2),
        in_specs=[
            pl.BlockSpec((_ROW_BLK, 128), lambda i, c: (i, 0)),
            pl.BlockSpec((_ROW_BLK, 128), lambda i, c: (nblk + i, 0)),
            pl.BlockSpec((_ROW_BLK, 1), lambda i, c: (i, 0)),
            pl.BlockSpec((128, 128), lambda i, c: (c, 0)),
            pl.BlockSpec((128, 128), lambda i, c: (c, 1)),
            pl.BlockSpec((1, 128), lambda i, c: (0, c)),
        ],
        out_specs=out_spec,
        out_shape=out_shape,
    )(aggAB, aggAB, rdeg2d, w, w, b2d)


def _mlp_body(x_ref, w1_ref, b1_ref, w2_ref, o_ref):
    h2 = jax.lax.dot_general(
        x_ref[...], w1_ref[...], (((1,), (1,)), ((), ())),
        preferred_element_type=jnp.float32,
        precision=jax.lax.Precision.HIGHEST)
    h2 = jnp.maximum(h2 + b1_ref[...], 0.0)
    o_ref[...] = jax.lax.dot_general(
        h2, w2_ref[...], (((1,), (1,)), ((), ())),
        preferred_element_type=jnp.float32,
        precision=jax.lax.Precision.HIGHEST)


def _mlp(xcat, fc1W, fc1b2d, fc2W):
    g = xcat.shape[0]
    return pl.pallas_call(
        _mlp_body,
        out_shape=jax.ShapeDtypeStruct((g, 1), jnp.float32),
    )(xcat, fc1W, fc1b2d, fc2W)


def kernel(x, edge_index, edge_weight, batch, emb, convW, convB, fc1W, fc1b, fc2W, fc2b):
    n, hdim = emb.shape
    e = edge_weight.shape[0]
    epad = ((e + _EBLK * _SUBCORES - 1) // (_EBLK * _SUBCORES)) * (_EBLK * _SUBCORES)

    src = edge_index[1]
    dst = edge_index[0]
    pad = epad - e
    src2d = jnp.concatenate([src, jnp.zeros((pad,), src.dtype)]).reshape(-1, 128)
    dst2d = jnp.concatenate(
        [dst, jnp.full((pad,), _NPAD - 1, dst.dtype)]).reshape(-1, 128)
    ew2d = jnp.concatenate(
        [edge_weight, jnp.zeros((pad,), edge_weight.dtype)]).reshape(-1, 128)

    embAB = emb.reshape(n, 2, 128).transpose(1, 0, 2).reshape(2 * n, 128)

    hAB = _emb_gather(embAB, x)
    deg2 = _deg_hist(dst2d).reshape(2, 128 * 128)
    deg = (deg2[0, :n] + deg2[1, :n])[:, None]
    rdeg2d = 1.0 / jnp.maximum(deg, 1.0)
    for i in range(convW.shape[0]):
        last = i == convW.shape[0] - 1
        aggAB = _msg_pass(hAB, src2d, dst2d, ew2d)
        hAB = _fused_layer(aggAB, rdeg2d, convW[i], convB[i][None, :], last)

    hfin = hAB
    ones = jnp.ones((n, 1), jnp.float32)
    counts = jax.ops.segment_sum(ones, batch, num_segments=_NUM_GRAPHS)
    sums = jax.ops.segment_sum(hfin, batch, num_segments=_NUM_GRAPHS)
    mean_x = sums / jnp.maximum(counts, 1.0)
    max_x = jax.ops.segment_max(hfin, batch, num_segments=_NUM_GRAPHS)
    max_x = jnp.where(counts > 0, max_x, 0.0)
    xcat = jnp.concatenate([mean_x, max_x], axis=1)

    out = _mlp(xcat, fc1W, fc1b[None, :], fc2W)
    return out[:, 0] + fc2b[0]
